# baseline (speedup 1.0000x reference)
import jax
import jax.numpy as jnp
import numpy as np
from jax.experimental import pallas as pl
from jax.experimental.pallas import tpu as pltpu

_NC = 20
_NA = 5
_NB = 64
_NH = 38
_NW = 38
_MAXB = 50
_THRESH = 0.6
_POS = _NA * _NH * _NW
_PPAD = 7680
_ROWS = _PPAD // 128

_P = np.arange(_PPAD)
_A = np.minimum(_P // (_NH * _NW), _NA - 1)
_S = _P % (_NH * _NW)
_VALID = (_P < _POS)
_COL = ((_S % _NW) * _VALID).astype(np.float32).reshape(_ROWS, 128)
_ROW = ((_S // _NW) * _VALID).astype(np.float32).reshape(_ROWS, 128)
_FIOTA = np.where(_VALID, _P, -1).astype(np.int32).reshape(_ROWS, 128)


def _region_loss_kernel(out_ref, tgt_ref, anc_ref, fio_ref, col_ref, row_ref,
                        awm_ref, ahm_ref, o_ref):
    f32 = jnp.float32
    x = jax.nn.sigmoid(out_ref[0, 0])
    y = jax.nn.sigmoid(out_ref[0, 1])
    w = out_ref[0, 2]
    h = out_ref[0, 3]
    conf = jax.nn.sigmoid(out_ref[0, 4])
    px = x + col_ref[:]
    py = y + row_ref[:]
    pw = jnp.exp(w) * awm_ref[:]
    ph = jnp.exp(h) * ahm_ref[:]
    pa = pw * ph
    fio = fio_ref[:]
    zero = jnp.zeros_like(x)

    def gt_cond(c):
        g = c[0]
        return jnp.logical_and(g < _MAXB, tgt_ref[0, 0, 5 * g + 1] != 0.0)

    def gt_body(c):
        g, mxi, mat, tcf, lg, ax, ay, aw_, ah_, sacc = c
        txg = tgt_ref[0, 0, 5 * g + 1]
        gx = txg * _NW
        gy = tgt_ref[0, 0, 5 * g + 2] * _NH
        gw = tgt_ref[0, 0, 5 * g + 3] * _NW
        gh = tgt_ref[0, 0, 5 * g + 4] * _NH
        cls = tgt_ref[0, 0, 5 * g].astype(jnp.int32)
        gi = jnp.clip(gx.astype(jnp.int32), 0, _NW - 1)
        gj = jnp.clip(gy.astype(jnp.int32), 0, _NH - 1)
        tx = gx - gi.astype(f32)
        ty = gy - gj.astype(f32)
        ga = gw * gh
        bi = jnp.minimum(anc_ref[0, 0], gw) * jnp.minimum(anc_ref[0, 1], gh)
        bu = anc_ref[0, 0] * anc_ref[0, 1] + ga - bi
        bn = jnp.int32(0)
        for n in range(1, _NA):
            i_n = jnp.minimum(anc_ref[0, 2 * n], gw) * \
                jnp.minimum(anc_ref[0, 2 * n + 1], gh)
            u_n = anc_ref[0, 2 * n] * anc_ref[0, 2 * n + 1] + ga - i_n
            better = i_n * bu > bi * u_n
            bn = jnp.where(better, jnp.int32(n), bn)
            bi = jnp.where(better, i_n, bi)
            bu = jnp.where(better, u_n, bu)
        awb = anc_ref[0, 2 * bn]
        ahb = anc_ref[0, 2 * bn + 1]
        num = jnp.concatenate(
            [jnp.full((8, 64), gw, f32), jnp.full((8, 64), gh, f32)], axis=1)
        den = jnp.concatenate(
            [jnp.full((8, 64), awb, f32), jnp.full((8, 64), ahb, f32)],
            axis=1)
        lgv = jnp.log(num / den)
        tw = lgv[0, 0]
        th = lgv[0, 64]
        p = bn * (_NH * _NW) + gj * _NW + gi
        mask = fio == p
        hw = gw * 0.5
        hh = gh * 0.5
        mnx = jnp.minimum(px - pw * 0.5, gx - hw)
        mxx = jnp.maximum(px + pw * 0.5, gx + hw)
        mny = jnp.minimum(py - ph * 0.5, gy - hh)
        mxy = jnp.maximum(py + ph * 0.5, gy + hh)
        cw = pw + gw - (mxx - mnx)
        ch = ph + gh - (mxy - mny)
        inter = jnp.where((cw <= 0.0) | (ch <= 0.0), 0.0, cw * ch)
        union = pa + ga - inter
        iou = inter / union
        cx = 0.5 - tx
        cy = 0.5 - ty
        sacc = sacc + (0.5 * (tw * tw + th * th)
                       - 0.5 * (cx * (tx + 0.5) + cy * (ty + 0.5)))
        return (g + 1,
                jnp.maximum(mxi, iou),
                jnp.where(mask, 1.0, mat),
                tcf + jnp.where(mask, iou, zero),
                lg + jnp.where(mask, out_ref[0, 5 + cls], zero),
                ax + jnp.where(mask, cx, 0.0),
                ay + jnp.where(mask, cy, 0.0),
                aw_ + jnp.where(mask, tw, 0.0),
                ah_ + jnp.where(mask, th, 0.0),
                sacc)

    init = (jnp.int32(0), zero, zero, zero, zero, zero, zero, zero, zero,
            jnp.float32(0.0))
    (_, mxi, mat, tcf, lg, ax, ay, aw_, ah_, sacc) = jax.lax.while_loop(
        gt_cond, gt_body, init)

    m = out_ref[0, 5]
    for c in range(6, 5 + _NC):
        m = jnp.maximum(m, out_ref[0, c])
    se = jnp.exp(out_ref[0, 5] - m)
    for c in range(6, 5 + _NC):
        se = se + jnp.exp(out_ref[0, c] - m)
    lse = m + jnp.log(se)

    bxy = (x - 0.5) ** 2 + (y - 0.5) ** 2 + w * w + h * h
    bgc = jnp.where((mxi <= _THRESH) & (mat == 0.0) & (fio >= 0),
                    conf * conf, 0.0)
    big = (0.5 * (bxy + bgc)
           + ax * x + ay * y - aw_ * w - ah_ * h
           + mat * (2.5 * conf * conf + lse) - lg
           - 5.0 * conf * tcf + 2.5 * tcf * tcf)
    o_ref[0, 0, 0] = jnp.sum(big) + sacc


@jax.jit
def kernel(output, target, anchors):
    f32 = jnp.float32
    aw = anchors.reshape(_NA, 2)[:, 0]
    ah = anchors.reshape(_NA, 2)[:, 1]
    vmask = jnp.asarray(_VALID.astype(np.float32))
    awm = (aw[_A] * vmask).reshape(_ROWS, 128)
    ahm = (ah[_A] * vmask).reshape(_ROWS, 128)

    out_t = output.reshape(_NB, _NA, 5 + _NC, _NH * _NW)
    out_t = out_t.transpose(0, 2, 1, 3).reshape(_NB, 5 + _NC, _POS)
    out_t = jnp.pad(out_t, ((0, 0), (0, 0), (0, _PPAD - _POS)))
    out_t = out_t.reshape(_NB, 5 + _NC, _ROWS, 128)

    partials = pl.pallas_call(
        _region_loss_kernel,
        grid=(_NB,),
        in_specs=[
            pl.BlockSpec((1, 5 + _NC, _ROWS, 128), lambda b: (b, 0, 0, 0)),
            pl.BlockSpec((1, 1, 5 * _MAXB), lambda b: (b, 0, 0),
                         memory_space=pltpu.SMEM),
            pl.BlockSpec((1, 2 * _NA), lambda b: (0, 0),
                         memory_space=pltpu.SMEM),
            pl.BlockSpec((_ROWS, 128), lambda b: (0, 0)),
            pl.BlockSpec((_ROWS, 128), lambda b: (0, 0)),
            pl.BlockSpec((_ROWS, 128), lambda b: (0, 0)),
            pl.BlockSpec((_ROWS, 128), lambda b: (0, 0)),
            pl.BlockSpec((_ROWS, 128), lambda b: (0, 0)),
        ],
        out_specs=pl.BlockSpec((1, 1, 1), lambda b: (b, 0, 0),
                               memory_space=pltpu.SMEM),
        out_shape=jax.ShapeDtypeStruct((_NB, 1, 1), f32),
        compiler_params=pltpu.CompilerParams(
            dimension_semantics=("parallel",)),
    )(out_t, target.reshape(_NB, 1, 5 * _MAXB), anchors.reshape(1, 2 * _NA),
      jnp.asarray(_FIOTA), jnp.asarray(_COL), jnp.asarray(_ROW), awm, ahm)
    return jnp.sum(partials)


# speedup vs baseline: 1.1613x; 1.1613x over previous
import jax
import jax.numpy as jnp
import numpy as np
from jax.experimental import pallas as pl
from jax.experimental.pallas import tpu as pltpu

_NC = 20
_NA = 5
_NB = 64
_NH = 38
_NW = 38
_MAXB = 50
_THRESH = 0.6
_POS = _NA * _NH * _NW
_PPAD = 7680
_ROWS = _PPAD // 128

_P = np.arange(_PPAD)
_A = np.minimum(_P // (_NH * _NW), _NA - 1)
_S = _P % (_NH * _NW)
_VALID = (_P < _POS)
_COL = ((_S % _NW) * _VALID).astype(np.float32).reshape(_ROWS, 128)
_ROW = ((_S // _NW) * _VALID).astype(np.float32).reshape(_ROWS, 128)
_FIOTA = np.where(_VALID, _P, -1).astype(np.int32).reshape(_ROWS, 128)


def _region_loss_kernel(out_ref, tgt_ref, anc_ref, fio_ref, col_ref, row_ref,
                        awm_ref, ahm_ref, o_ref):
    f32 = jnp.float32
    x = jax.nn.sigmoid(out_ref[0, 0])
    y = jax.nn.sigmoid(out_ref[0, 1])
    w = out_ref[0, 2]
    h = out_ref[0, 3]
    conf = jax.nn.sigmoid(out_ref[0, 4])
    px = x + col_ref[:]
    py = y + row_ref[:]
    pw = jnp.exp(w) * awm_ref[:]
    ph = jnp.exp(h) * ahm_ref[:]
    pa = pw * ph
    fio = fio_ref[:]
    zero = jnp.zeros_like(x)

    def gt_cond(c):
        g = c[0]
        return jnp.logical_and(g < _MAXB, tgt_ref[0, 0, 5 * g + 1] != 0.0)

    def gt_body(c):
        g, mxi, mat, tcf, lg, ax, ay, aw_, ah_, sacc = c
        txg = tgt_ref[0, 0, 5 * g + 1]
        gx = txg * _NW
        gy = tgt_ref[0, 0, 5 * g + 2] * _NH
        gw = tgt_ref[0, 0, 5 * g + 3] * _NW
        gh = tgt_ref[0, 0, 5 * g + 4] * _NH
        cls = tgt_ref[0, 0, 5 * g].astype(jnp.int32)
        gi = jnp.clip(gx.astype(jnp.int32), 0, _NW - 1)
        gj = jnp.clip(gy.astype(jnp.int32), 0, _NH - 1)
        tx = gx - gi.astype(f32)
        ty = gy - gj.astype(f32)
        ga = gw * gh
        bi = jnp.minimum(anc_ref[0, 0], gw) * jnp.minimum(anc_ref[0, 1], gh)
        bu = anc_ref[0, 0] * anc_ref[0, 1] + ga - bi
        bn = jnp.int32(0)
        for n in range(1, _NA):
            i_n = jnp.minimum(anc_ref[0, 2 * n], gw) * \
                jnp.minimum(anc_ref[0, 2 * n + 1], gh)
            u_n = anc_ref[0, 2 * n] * anc_ref[0, 2 * n + 1] + ga - i_n
            better = i_n * bu > bi * u_n
            bn = jnp.where(better, jnp.int32(n), bn)
            bi = jnp.where(better, i_n, bi)
            bu = jnp.where(better, u_n, bu)
        awb = anc_ref[0, 2 * bn]
        ahb = anc_ref[0, 2 * bn + 1]
        num = jnp.concatenate(
            [jnp.full((8, 64), gw, f32), jnp.full((8, 64), gh, f32)], axis=1)
        den = jnp.concatenate(
            [jnp.full((8, 64), awb, f32), jnp.full((8, 64), ahb, f32)],
            axis=1)
        lgv = jnp.log(num / den)
        tw = lgv[0, 0]
        th = lgv[0, 64]
        p = bn * (_NH * _NW) + gj * _NW + gi
        mask = fio == p
        hw = gw * 0.5
        hh = gh * 0.5
        mnx = jnp.minimum(px - pw * 0.5, gx - hw)
        mxx = jnp.maximum(px + pw * 0.5, gx + hw)
        mny = jnp.minimum(py - ph * 0.5, gy - hh)
        mxy = jnp.maximum(py + ph * 0.5, gy + hh)
        cw = pw + gw - (mxx - mnx)
        ch = ph + gh - (mxy - mny)
        inter = jnp.where((cw <= 0.0) | (ch <= 0.0), 0.0, cw * ch)
        union = pa + ga - inter
        iou = inter / union
        cx = 0.5 - tx
        cy = 0.5 - ty
        sacc = sacc + (0.5 * (tw * tw + th * th)
                       - 0.5 * (cx * (tx + 0.5) + cy * (ty + 0.5)))
        return (g + 1,
                jnp.maximum(mxi, iou),
                jnp.where(mask, 1.0, mat),
                tcf + jnp.where(mask, iou, zero),
                lg + jnp.where(mask, out_ref[0, 5 + cls], zero),
                ax + jnp.where(mask, cx, 0.0),
                ay + jnp.where(mask, cy, 0.0),
                aw_ + jnp.where(mask, tw, 0.0),
                ah_ + jnp.where(mask, th, 0.0),
                sacc)

    init = (jnp.int32(0), zero, zero, zero, zero, zero, zero, zero, zero,
            jnp.float32(0.0))
    (_, mxi, mat, tcf, lg, ax, ay, aw_, ah_, sacc) = init

    m = out_ref[0, 5]
    for c in range(6, 5 + _NC):
        m = jnp.maximum(m, out_ref[0, c])
    se = jnp.exp(out_ref[0, 5] - m)
    for c in range(6, 5 + _NC):
        se = se + jnp.exp(out_ref[0, c] - m)
    lse = m + jnp.log(se)

    bxy = (x - 0.5) ** 2 + (y - 0.5) ** 2 + w * w + h * h
    bgc = jnp.where((mxi <= _THRESH) & (mat == 0.0) & (fio >= 0),
                    conf * conf, 0.0)
    big = (0.5 * (bxy + bgc)
           + ax * x + ay * y - aw_ * w - ah_ * h
           + mat * (2.5 * conf * conf + lse) - lg
           - 5.0 * conf * tcf + 2.5 * tcf * tcf)
    o_ref[0, 0, 0] = jnp.sum(big) + sacc


@jax.jit
def kernel(output, target, anchors):
    f32 = jnp.float32
    aw = anchors.reshape(_NA, 2)[:, 0]
    ah = anchors.reshape(_NA, 2)[:, 1]
    vmask = jnp.asarray(_VALID.astype(np.float32))
    awm = (aw[_A] * vmask).reshape(_ROWS, 128)
    ahm = (ah[_A] * vmask).reshape(_ROWS, 128)

    out_t = output.reshape(_NB, _NA, 5 + _NC, _NH * _NW)
    out_t = out_t.transpose(0, 2, 1, 3).reshape(_NB, 5 + _NC, _POS)
    out_t = jnp.pad(out_t, ((0, 0), (0, 0), (0, _PPAD - _POS)))
    out_t = out_t.reshape(_NB, 5 + _NC, _ROWS, 128)

    partials = pl.pallas_call(
        _region_loss_kernel,
        grid=(_NB,),
        in_specs=[
            pl.BlockSpec((1, 5 + _NC, _ROWS, 128), lambda b: (b, 0, 0, 0)),
            pl.BlockSpec((1, 1, 5 * _MAXB), lambda b: (b, 0, 0),
                         memory_space=pltpu.SMEM),
            pl.BlockSpec((1, 2 * _NA), lambda b: (0, 0),
                         memory_space=pltpu.SMEM),
            pl.BlockSpec((_ROWS, 128), lambda b: (0, 0)),
            pl.BlockSpec((_ROWS, 128), lambda b: (0, 0)),
            pl.BlockSpec((_ROWS, 128), lambda b: (0, 0)),
            pl.BlockSpec((_ROWS, 128), lambda b: (0, 0)),
            pl.BlockSpec((_ROWS, 128), lambda b: (0, 0)),
        ],
        out_specs=pl.BlockSpec((1, 1, 1), lambda b: (b, 0, 0),
                               memory_space=pltpu.SMEM),
        out_shape=jax.ShapeDtypeStruct((_NB, 1, 1), f32),
        compiler_params=pltpu.CompilerParams(
            dimension_semantics=("parallel",)),
    )(out_t, target.reshape(_NB, 1, 5 * _MAXB), anchors.reshape(1, 2 * _NA),
      jnp.asarray(_FIOTA), jnp.asarray(_COL), jnp.asarray(_ROW), awm, ahm)
    return jnp.sum(partials)


# speedup vs baseline: 2.9955x; 2.5795x over previous
import jax
import jax.numpy as jnp
import numpy as np
from jax.experimental import pallas as pl
from jax.experimental.pallas import tpu as pltpu

_NC = 20
_NA = 5
_NB = 64
_NH = 38
_NW = 38
_MAXB = 50
_THRESH = 0.6
_SR = 4
_SL = 361

_S = np.arange(_NH * _NW).reshape(1, _SR, _SL) + np.zeros((_NA, 1, 1), int)
_AIDX = np.arange(_NA).reshape(_NA, 1, 1) + np.zeros((1, _SR, _SL), int)
_COL = (_S % _NW).astype(np.float32)
_ROW = (_S // _NW).astype(np.float32)
_FIOTA = (_AIDX * (_NH * _NW) + _S).astype(np.int32)


def _region_loss_kernel(out_ref, tgt_ref, anc_ref, fio_ref, col_ref, row_ref,
                        awm_ref, ahm_ref, o_ref):
    f32 = jnp.float32

    def ch(c):
        return jnp.stack([out_ref[0, 25 * a + c] for a in range(_NA)])

    x = jax.nn.sigmoid(ch(0))
    y = jax.nn.sigmoid(ch(1))
    w = ch(2)
    h = ch(3)
    conf = jax.nn.sigmoid(ch(4))
    px = x + col_ref[:]
    py = y + row_ref[:]
    pw = jnp.exp(w) * awm_ref[:]
    ph = jnp.exp(h) * ahm_ref[:]
    pa = pw * ph
    fio = fio_ref[:]
    zero = jnp.zeros_like(x)

    def gt_cond(c):
        g = c[0]
        return jnp.logical_and(g < _MAXB, tgt_ref[0, 0, 5 * g + 1] != 0.0)

    def gt_body(c):
        g, mxi, mat, tcf, lg, ax, ay, aw_, ah_, sacc = c
        txg = tgt_ref[0, 0, 5 * g + 1]
        gx = txg * _NW
        gy = tgt_ref[0, 0, 5 * g + 2] * _NH
        gw = tgt_ref[0, 0, 5 * g + 3] * _NW
        gh = tgt_ref[0, 0, 5 * g + 4] * _NH
        cls = tgt_ref[0, 0, 5 * g].astype(jnp.int32)
        gi = jnp.clip(gx.astype(jnp.int32), 0, _NW - 1)
        gj = jnp.clip(gy.astype(jnp.int32), 0, _NH - 1)
        tx = gx - gi.astype(f32)
        ty = gy - gj.astype(f32)
        ga = gw * gh
        bi = jnp.minimum(anc_ref[0, 0], gw) * jnp.minimum(anc_ref[0, 1], gh)
        bu = anc_ref[0, 0] * anc_ref[0, 1] + ga - bi
        bn = jnp.int32(0)
        for n in range(1, _NA):
            i_n = jnp.minimum(anc_ref[0, 2 * n], gw) * \
                jnp.minimum(anc_ref[0, 2 * n + 1], gh)
            u_n = anc_ref[0, 2 * n] * anc_ref[0, 2 * n + 1] + ga - i_n
            better = i_n * bu > bi * u_n
            bn = jnp.where(better, jnp.int32(n), bn)
            bi = jnp.where(better, i_n, bi)
            bu = jnp.where(better, u_n, bu)
        awb = anc_ref[0, 2 * bn]
        ahb = anc_ref[0, 2 * bn + 1]
        num = jnp.concatenate(
            [jnp.full((8, 64), gw, f32), jnp.full((8, 64), gh, f32)], axis=1)
        den = jnp.concatenate(
            [jnp.full((8, 64), awb, f32), jnp.full((8, 64), ahb, f32)],
            axis=1)
        lgv = jnp.log(num / den)
        tw = lgv[0, 0]
        th = lgv[0, 64]
        p = bn * (_NH * _NW) + gj * _NW + gi
        mask = fio == p
        hw = gw * 0.5
        hh = gh * 0.5
        mnx = jnp.minimum(px - pw * 0.5, gx - hw)
        mxx = jnp.maximum(px + pw * 0.5, gx + hw)
        mny = jnp.minimum(py - ph * 0.5, gy - hh)
        mxy = jnp.maximum(py + ph * 0.5, gy + hh)
        cw = pw + gw - (mxx - mnx)
        ch_ = ph + gh - (mxy - mny)
        inter = jnp.where((cw <= 0.0) | (ch_ <= 0.0), 0.0, cw * ch_)
        union = pa + ga - inter
        iou = inter / union
        cx = 0.5 - tx
        cy = 0.5 - ty
        lgm = jnp.stack(
            [out_ref[0, 25 * a + 5 + cls] for a in range(_NA)])
        sacc = sacc + (0.5 * (tw * tw + th * th)
                       - 0.5 * (cx * (tx + 0.5) + cy * (ty + 0.5)))
        return (g + 1,
                jnp.maximum(mxi, iou),
                jnp.where(mask, 1.0, mat),
                tcf + jnp.where(mask, iou, zero),
                lg + jnp.where(mask, lgm, zero),
                ax + jnp.where(mask, cx, 0.0),
                ay + jnp.where(mask, cy, 0.0),
                aw_ + jnp.where(mask, tw, 0.0),
                ah_ + jnp.where(mask, th, 0.0),
                sacc)

    init = (jnp.int32(0), zero, zero, zero, zero, zero, zero, zero, zero,
            jnp.float32(0.0))
    (_, mxi, mat, tcf, lg, ax, ay, aw_, ah_, sacc) = jax.lax.while_loop(
        gt_cond, gt_body, init)

    m = ch(5)
    for c in range(6, 5 + _NC):
        m = jnp.maximum(m, ch(c))
    se = jnp.exp(ch(5) - m)
    for c in range(6, 5 + _NC):
        se = se + jnp.exp(ch(c) - m)
    lse = m + jnp.log(se)

    bxy = (x - 0.5) ** 2 + (y - 0.5) ** 2 + w * w + h * h
    bgc = jnp.where((mxi <= _THRESH) & (mat == 0.0), conf * conf, 0.0)
    big = (0.5 * (bxy + bgc)
           + ax * x + ay * y - aw_ * w - ah_ * h
           + mat * (2.5 * conf * conf + lse) - lg
           - 5.0 * conf * tcf + 2.5 * tcf * tcf)
    o_ref[0, 0, 0] = jnp.sum(big) + sacc


@jax.jit
def kernel(output, target, anchors):
    f32 = jnp.float32
    aw = anchors.reshape(_NA, 2)[:, 0]
    ah = anchors.reshape(_NA, 2)[:, 1]
    awm = jnp.broadcast_to(aw[:, None, None], (_NA, _SR, _SL))
    ahm = jnp.broadcast_to(ah[:, None, None], (_NA, _SR, _SL))

    out_n = output.reshape(_NB, _NA * (5 + _NC), _SR, _SL)

    partials = pl.pallas_call(
        _region_loss_kernel,
        grid=(_NB,),
        in_specs=[
            pl.BlockSpec((1, _NA * (5 + _NC), _SR, _SL),
                         lambda b: (b, 0, 0, 0)),
            pl.BlockSpec((1, 1, 5 * _MAXB), lambda b: (b, 0, 0),
                         memory_space=pltpu.SMEM),
            pl.BlockSpec((1, 2 * _NA), lambda b: (0, 0),
                         memory_space=pltpu.SMEM),
            pl.BlockSpec((_NA, _SR, _SL), lambda b: (0, 0, 0)),
            pl.BlockSpec((_NA, _SR, _SL), lambda b: (0, 0, 0)),
            pl.BlockSpec((_NA, _SR, _SL), lambda b: (0, 0, 0)),
            pl.BlockSpec((_NA, _SR, _SL), lambda b: (0, 0, 0)),
            pl.BlockSpec((_NA, _SR, _SL), lambda b: (0, 0, 0)),
        ],
        out_specs=pl.BlockSpec((1, 1, 1), lambda b: (b, 0, 0),
                               memory_space=pltpu.SMEM),
        out_shape=jax.ShapeDtypeStruct((_NB, 1, 1), f32),
        compiler_params=pltpu.CompilerParams(
            dimension_semantics=("parallel",)),
    )(out_n, target.reshape(_NB, 1, 5 * _MAXB), anchors.reshape(1, 2 * _NA),
      jnp.asarray(_FIOTA), jnp.asarray(_COL), jnp.asarray(_ROW), awm, ahm)
    return jnp.sum(partials)
